# baseline (speedup 1.0000x reference)
import functools

import jax
import jax.numpy as jnp
from jax import lax
from jax.experimental import pallas as pl
from jax.experimental.pallas import tpu as pltpu
from jax.experimental.pallas import tpu_sc as plsc

BATCH = 16384
NUM_FIELDS = 26
NUM_WORKERS = 32
ROWS_PER_W = BATCH // NUM_WORKERS
IDX_PER_W = ROWS_PER_W * NUM_FIELDS


@functools.partial(
    pl.kernel,
    out_type=jax.ShapeDtypeStruct((BATCH,), jnp.float32),
    mesh=plsc.VectorSubcoreMesh(core_axis_name="c", subcore_axis_name="s"),
    compiler_params=pltpu.CompilerParams(needs_layout_passes=False),
    scratch_types=[
        pltpu.VMEM((IDX_PER_W,), jnp.int32),
        pltpu.VMEM((IDX_PER_W,), jnp.float32),
        pltpu.VMEM((ROWS_PER_W,), jnp.float32),
        pltpu.SemaphoreType.DMA,
    ],
)
def _emb_sum(x_hbm, table_hbm, out_hbm, idx_v, vals_v, out_v, sem):
    wid = lax.axis_index("s") * 2 + lax.axis_index("c")

    pltpu.sync_copy(x_hbm.at[pl.ds(wid * IDX_PER_W, IDX_PER_W)], idx_v)
    pltpu.async_copy(table_hbm.at[idx_v], vals_v, sem).wait()

    lane26 = lax.iota(jnp.int32, 16) * NUM_FIELDS

    def accum(i, _):
        base = lane26 + i * (16 * NUM_FIELDS)

        def fbody(f, acc):
            return acc + plsc.load_gather(vals_v, [base + f])

        acc = lax.fori_loop(0, NUM_FIELDS, fbody, jnp.zeros((16,), jnp.float32))
        out_v[pl.ds(i * 16, 16)] = acc
        return 0

    lax.fori_loop(0, ROWS_PER_W // 16, accum, 0)
    pltpu.sync_copy(out_v, out_hbm.at[pl.ds(wid * ROWS_PER_W, ROWS_PER_W)])


def kernel(x, table, bias):
    out = _emb_sum(x.reshape(-1), table.reshape(-1))
    return out.reshape(BATCH, 1) + bias


# speedup vs baseline: 2.8015x; 2.8015x over previous
import functools

import jax
import jax.numpy as jnp
from jax import lax
from jax.experimental import pallas as pl
from jax.experimental.pallas import tpu as pltpu
from jax.experimental.pallas import tpu_sc as plsc

BATCH = 16384
NUM_FIELDS = 26
NUM_EMB = 2600000
TBL_SPLIT = 2599936
NUM_WORKERS = 32
ROWS_PER_W = BATCH // NUM_WORKERS
IDX_PER_W = ROWS_PER_W * NUM_FIELDS


@functools.partial(
    pl.kernel,
    out_type=jax.ShapeDtypeStruct((BATCH,), jnp.float32),
    mesh=plsc.VectorSubcoreMesh(core_axis_name="c", subcore_axis_name="s"),
    scratch_types=[
        pltpu.VMEM((IDX_PER_W,), jnp.int32),
        pltpu.VMEM((IDX_PER_W,), jnp.float32),
        pltpu.VMEM((ROWS_PER_W,), jnp.float32),
        pltpu.SemaphoreType.DMA,
    ],
)
def _emb_sum(x_hbm, table_hbm, out_hbm, idx_v, vals_v, out_v, sem):
    wid = lax.axis_index("s") * 2 + lax.axis_index("c")

    pltpu.sync_copy(x_hbm.at[pl.ds(wid * IDX_PER_W, IDX_PER_W)], idx_v)
    pltpu.async_copy(table_hbm.at[idx_v], vals_v, sem).wait()

    def accum(i, _):
        def fbody(f, acc):
            return acc + vals_v[pl.ds(f * ROWS_PER_W + i * 16, 16)]

        acc = lax.fori_loop(0, NUM_FIELDS, fbody, jnp.zeros((16,), jnp.float32))
        out_v[pl.ds(i * 16, 16)] = acc
        return 0

    lax.fori_loop(0, ROWS_PER_W // 16, accum, 0)
    pltpu.sync_copy(out_v, out_hbm.at[pl.ds(wid * ROWS_PER_W, ROWS_PER_W)])


def kernel(x, table, bias):
    table_lin = jnp.concatenate(
        [
            lax.optimization_barrier(table[:TBL_SPLIT]).reshape(-1),
            table[TBL_SPLIT:].reshape(-1),
        ]
    )
    xw = (
        x.T.reshape(NUM_FIELDS, NUM_WORKERS, ROWS_PER_W)
        .transpose(1, 0, 2)
        .reshape(NUM_WORKERS * IDX_PER_W)
    )
    out = _emb_sum(xw, table_lin)
    return out.reshape(BATCH, 1) + bias


# speedup vs baseline: 3.2898x; 1.1743x over previous
import functools

import jax
import jax.numpy as jnp
from jax import lax
from jax.experimental import pallas as pl
from jax.experimental.pallas import tpu as pltpu
from jax.experimental.pallas import tpu_sc as plsc

BATCH = 16384
NUM_FIELDS = 26
NUM_EMB = 2600000
TBL_SPLIT = 2599936
NUM_WORKERS = 32
ROWS_PER_W = BATCH // NUM_WORKERS
IDX_PER_W = ROWS_PER_W * NUM_FIELDS
F_HALF = NUM_FIELDS // 2
IDX_HALF = F_HALF * ROWS_PER_W


@functools.partial(
    pl.kernel,
    out_type=jax.ShapeDtypeStruct((BATCH,), jnp.float32),
    mesh=plsc.VectorSubcoreMesh(core_axis_name="c", subcore_axis_name="s"),
    scratch_types=[
        pltpu.VMEM((IDX_PER_W,), jnp.int32),
        pltpu.VMEM((IDX_PER_W,), jnp.float32),
        pltpu.VMEM((ROWS_PER_W,), jnp.float32),
        pltpu.VMEM((16,), jnp.float32),
        pltpu.SemaphoreType.DMA,
        pltpu.SemaphoreType.DMA,
    ],
)
def _emb_sum(x_hbm, table_hbm, bias_hbm, out_hbm, idx_v, vals_v, out_v, bias_v,
             sem1, sem2):
    wid = lax.axis_index("s") * 2 + lax.axis_index("c")

    pltpu.sync_copy(x_hbm.at[wid], idx_v)
    pltpu.sync_copy(bias_hbm, bias_v)
    g1 = pltpu.async_copy(
        table_hbm.at[idx_v.at[pl.ds(0, IDX_HALF)]],
        vals_v.at[pl.ds(0, IDX_HALF)],
        sem1,
    )
    g2 = pltpu.async_copy(
        table_hbm.at[idx_v.at[pl.ds(IDX_HALF, IDX_PER_W - IDX_HALF)]],
        vals_v.at[pl.ds(IDX_HALF, IDX_PER_W - IDX_HALF)],
        sem2,
    )
    bias_vec = bias_v[...]
    g1.wait()

    def accum1(i, _):
        acc = bias_vec
        for f in range(F_HALF):
            acc = acc + vals_v[pl.ds(f * ROWS_PER_W + i * 16, 16)]
        out_v[pl.ds(i * 16, 16)] = acc
        return 0

    lax.fori_loop(0, ROWS_PER_W // 16, accum1, 0)
    g2.wait()

    def accum2(i, _):
        acc = out_v[pl.ds(i * 16, 16)]
        for f in range(F_HALF, NUM_FIELDS):
            acc = acc + vals_v[pl.ds(f * ROWS_PER_W + i * 16, 16)]
        out_v[pl.ds(i * 16, 16)] = acc
        return 0

    lax.fori_loop(0, ROWS_PER_W // 16, accum2, 0)
    pltpu.sync_copy(out_v, out_hbm.at[pl.ds(wid * ROWS_PER_W, ROWS_PER_W)])


def kernel(x, table, bias):
    table_lin = jnp.concatenate(
        [
            lax.optimization_barrier(table[:TBL_SPLIT]).reshape(-1),
            table[TBL_SPLIT:].reshape(-1),
        ]
    )
    xw = (
        x.T.reshape(NUM_FIELDS, NUM_WORKERS, ROWS_PER_W)
        .transpose(1, 0, 2)
        .reshape(NUM_WORKERS, IDX_PER_W)
    )
    out = _emb_sum(xw, table_lin, jnp.broadcast_to(bias, (16,)))
    return out.reshape(BATCH, 1)


# speedup vs baseline: 3.3568x; 1.0204x over previous
import functools

import jax
import jax.numpy as jnp
from jax import lax
from jax.experimental import pallas as pl
from jax.experimental.pallas import tpu as pltpu
from jax.experimental.pallas import tpu_sc as plsc

BATCH = 16384
NUM_FIELDS = 26
NUM_EMB = 2600000
TBL_SPLIT = 2599936
NUM_WORKERS = 32
ROWS_PER_W = BATCH // NUM_WORKERS
IDX_PER_W = ROWS_PER_W * NUM_FIELDS
F_HALF = NUM_FIELDS // 2
IDX_HALF = F_HALF * ROWS_PER_W


@functools.partial(
    pl.kernel,
    out_type=jax.ShapeDtypeStruct((BATCH,), jnp.float32),
    mesh=plsc.VectorSubcoreMesh(core_axis_name="c", subcore_axis_name="s"),
    scratch_types=[
        pltpu.VMEM((IDX_PER_W,), jnp.int32),
        pltpu.VMEM((IDX_PER_W,), jnp.float32),
        pltpu.VMEM((ROWS_PER_W,), jnp.float32),
        pltpu.VMEM((16,), jnp.float32),
        pltpu.SemaphoreType.DMA,
        pltpu.SemaphoreType.DMA,
    ],
)
def _emb_sum(x_hbm, table_hbm, bias_hbm, out_hbm, idx_v, vals_v, out_v, bias_v,
             sem1, sem2):
    wid = lax.axis_index("s") * 2 + lax.axis_index("c")

    pltpu.sync_copy(x_hbm.at[wid, pl.ds(0, IDX_HALF)], idx_v.at[pl.ds(0, IDX_HALF)])
    g1 = pltpu.async_copy(
        table_hbm.at[idx_v.at[pl.ds(0, IDX_HALF)]],
        vals_v.at[pl.ds(0, IDX_HALF)],
        sem1,
    )
    pltpu.sync_copy(
        x_hbm.at[wid, pl.ds(IDX_HALF, IDX_PER_W - IDX_HALF)],
        idx_v.at[pl.ds(IDX_HALF, IDX_PER_W - IDX_HALF)],
    )
    pltpu.sync_copy(bias_hbm, bias_v)
    g2 = pltpu.async_copy(
        table_hbm.at[idx_v.at[pl.ds(IDX_HALF, IDX_PER_W - IDX_HALF)]],
        vals_v.at[pl.ds(IDX_HALF, IDX_PER_W - IDX_HALF)],
        sem2,
    )
    bias_vec = bias_v[...]
    g1.wait()

    def accum1(i, _):
        acc0 = bias_vec
        acc1 = vals_v[pl.ds(i * 16, 16)]
        for f in range(1, F_HALF, 2):
            acc0 = acc0 + vals_v[pl.ds(f * ROWS_PER_W + i * 16, 16)]
        for f in range(2, F_HALF, 2):
            acc1 = acc1 + vals_v[pl.ds(f * ROWS_PER_W + i * 16, 16)]
        out_v[pl.ds(i * 16, 16)] = acc0 + acc1
        return 0

    lax.fori_loop(0, ROWS_PER_W // 16, accum1, 0)
    g2.wait()

    def accum2(i, _):
        acc0 = out_v[pl.ds(i * 16, 16)]
        acc1 = vals_v[pl.ds(F_HALF * ROWS_PER_W + i * 16, 16)]
        for f in range(F_HALF + 1, NUM_FIELDS, 2):
            acc0 = acc0 + vals_v[pl.ds(f * ROWS_PER_W + i * 16, 16)]
        for f in range(F_HALF + 2, NUM_FIELDS, 2):
            acc1 = acc1 + vals_v[pl.ds(f * ROWS_PER_W + i * 16, 16)]
        out_v[pl.ds(i * 16, 16)] = acc0 + acc1
        return 0

    lax.fori_loop(0, ROWS_PER_W // 16, accum2, 0)
    pltpu.sync_copy(out_v, out_hbm.at[pl.ds(wid * ROWS_PER_W, ROWS_PER_W)])


def kernel(x, table, bias):
    table_lin = jnp.concatenate(
        [
            lax.optimization_barrier(table[:TBL_SPLIT]).reshape(-1),
            table[TBL_SPLIT:].reshape(-1),
        ]
    )
    xw = (
        x.T.reshape(NUM_FIELDS, NUM_WORKERS, ROWS_PER_W)
        .transpose(1, 0, 2)
        .reshape(NUM_WORKERS, IDX_PER_W)
    )
    out = _emb_sum(xw, table_lin, jnp.broadcast_to(bias, (16,)))
    return out.reshape(BATCH, 1)


# speedup vs baseline: 3.4690x; 1.0334x over previous
import functools

import jax
import jax.numpy as jnp
from jax import lax
from jax.experimental import pallas as pl
from jax.experimental.pallas import tpu as pltpu
from jax.experimental.pallas import tpu_sc as plsc

BATCH = 16384
NUM_FIELDS = 26
NUM_EMB = 2600000
TBL_SPLIT = 2599936
NUM_WORKERS = 32
ROWS_PER_W = BATCH // NUM_WORKERS
IDX_PER_W = ROWS_PER_W * NUM_FIELDS
F_HALF = NUM_FIELDS // 2
IDX_HALF = F_HALF * ROWS_PER_W


@functools.partial(
    pl.kernel,
    out_type=jax.ShapeDtypeStruct((BATCH,), jnp.float32),
    mesh=plsc.VectorSubcoreMesh(core_axis_name="c", subcore_axis_name="s"),
    scratch_types=[
        pltpu.VMEM((IDX_PER_W,), jnp.int32),
        pltpu.VMEM((IDX_PER_W,), jnp.float32),
        pltpu.VMEM((ROWS_PER_W,), jnp.float32),
        pltpu.VMEM((16,), jnp.float32),
        pltpu.SemaphoreType.DMA,
        pltpu.SemaphoreType.DMA,
        pltpu.SemaphoreType.DMA,
    ],
)
def _emb_sum(x_hbm, table_hbm, bias_hbm, out_hbm, idx_v, vals_v, out_v, bias_v,
             sem1, sem2, sem3):
    wid = lax.axis_index("s") * 2 + lax.axis_index("c")

    col = wid * ROWS_PER_W
    for f in range(F_HALF):
        pltpu.async_copy(
            x_hbm.at[f, pl.ds(col, ROWS_PER_W)],
            idx_v.at[pl.ds(f * ROWS_PER_W, ROWS_PER_W)],
            sem3,
        )
    for f in range(F_HALF):
        pltpu.make_async_copy(
            x_hbm.at[f, pl.ds(col, ROWS_PER_W)],
            idx_v.at[pl.ds(f * ROWS_PER_W, ROWS_PER_W)],
            sem3,
        ).wait()
    g1 = pltpu.async_copy(
        table_hbm.at[idx_v.at[pl.ds(0, IDX_HALF)]],
        vals_v.at[pl.ds(0, IDX_HALF)],
        sem1,
    )
    for f in range(F_HALF, NUM_FIELDS):
        pltpu.async_copy(
            x_hbm.at[f, pl.ds(col, ROWS_PER_W)],
            idx_v.at[pl.ds(f * ROWS_PER_W, ROWS_PER_W)],
            sem3,
        )
    for f in range(F_HALF, NUM_FIELDS):
        pltpu.make_async_copy(
            x_hbm.at[f, pl.ds(col, ROWS_PER_W)],
            idx_v.at[pl.ds(f * ROWS_PER_W, ROWS_PER_W)],
            sem3,
        ).wait()
    pltpu.sync_copy(bias_hbm, bias_v)
    g2 = pltpu.async_copy(
        table_hbm.at[idx_v.at[pl.ds(IDX_HALF, IDX_PER_W - IDX_HALF)]],
        vals_v.at[pl.ds(IDX_HALF, IDX_PER_W - IDX_HALF)],
        sem2,
    )
    bias_vec = bias_v[...]
    g1.wait()

    def accum1(i, _):
        acc0 = bias_vec
        acc1 = vals_v[pl.ds(i * 16, 16)]
        for f in range(1, F_HALF, 2):
            acc0 = acc0 + vals_v[pl.ds(f * ROWS_PER_W + i * 16, 16)]
        for f in range(2, F_HALF, 2):
            acc1 = acc1 + vals_v[pl.ds(f * ROWS_PER_W + i * 16, 16)]
        out_v[pl.ds(i * 16, 16)] = acc0 + acc1
        return 0

    lax.fori_loop(0, ROWS_PER_W // 16, accum1, 0)
    g2.wait()

    def accum2(i, _):
        acc0 = out_v[pl.ds(i * 16, 16)]
        acc1 = vals_v[pl.ds(F_HALF * ROWS_PER_W + i * 16, 16)]
        for f in range(F_HALF + 1, NUM_FIELDS, 2):
            acc0 = acc0 + vals_v[pl.ds(f * ROWS_PER_W + i * 16, 16)]
        for f in range(F_HALF + 2, NUM_FIELDS, 2):
            acc1 = acc1 + vals_v[pl.ds(f * ROWS_PER_W + i * 16, 16)]
        out_v[pl.ds(i * 16, 16)] = acc0 + acc1
        return 0

    lax.fori_loop(0, ROWS_PER_W // 16, accum2, 0)
    pltpu.sync_copy(out_v, out_hbm.at[pl.ds(wid * ROWS_PER_W, ROWS_PER_W)])


def kernel(x, table, bias):
    table_lin = jnp.concatenate(
        [
            lax.optimization_barrier(table[:TBL_SPLIT]).reshape(-1),
            table[TBL_SPLIT:].reshape(-1),
        ]
    )
    xw = x.T
    out = _emb_sum(xw, table_lin, jnp.broadcast_to(bias, (16,)))
    return out.reshape(BATCH, 1)


# speedup vs baseline: 3.4723x; 1.0009x over previous
import functools

import jax
import jax.numpy as jnp
from jax import lax
from jax.experimental import pallas as pl
from jax.experimental.pallas import tpu as pltpu
from jax.experimental.pallas import tpu_sc as plsc

BATCH = 16384
NUM_FIELDS = 26
NUM_EMB = 2600000
TBL_SPLIT = 2599936
NUM_WORKERS = 32
ROWS_PER_W = BATCH // NUM_WORKERS
IDX_PER_W = ROWS_PER_W * NUM_FIELDS
F_HALF = NUM_FIELDS // 2
IDX_HALF = F_HALF * ROWS_PER_W


@functools.partial(
    pl.kernel,
    out_type=jax.ShapeDtypeStruct((BATCH,), jnp.float32),
    mesh=plsc.VectorSubcoreMesh(core_axis_name="c", subcore_axis_name="s"),
    scratch_types=[
        pltpu.VMEM((IDX_PER_W,), jnp.int32),
        pltpu.VMEM((IDX_PER_W,), jnp.float32),
        pltpu.VMEM((ROWS_PER_W,), jnp.float32),
        pltpu.VMEM((16,), jnp.float32),
        pltpu.SemaphoreType.DMA,
        pltpu.SemaphoreType.DMA,
        pltpu.SemaphoreType.DMA,
    ],
)
def _emb_sum(x_hbm, table_hbm, bias_hbm, out_hbm, idx_v, vals_v, out_v, bias_v,
             sem1, sem2, sem3):
    wid = lax.axis_index("s") * 2 + lax.axis_index("c")

    col = wid * ROWS_PER_W
    for f in range(F_HALF):
        pltpu.async_copy(
            x_hbm.at[f, pl.ds(col, ROWS_PER_W)],
            idx_v.at[pl.ds(f * ROWS_PER_W, ROWS_PER_W)],
            sem3,
        )
    for f in range(F_HALF):
        pltpu.make_async_copy(
            x_hbm.at[f, pl.ds(col, ROWS_PER_W)],
            idx_v.at[pl.ds(f * ROWS_PER_W, ROWS_PER_W)],
            sem3,
        ).wait()
    g1 = pltpu.async_copy(
        table_hbm.at[idx_v.at[pl.ds(0, IDX_HALF)]],
        vals_v.at[pl.ds(0, IDX_HALF)],
        sem1,
    )
    for f in range(F_HALF, NUM_FIELDS):
        pltpu.async_copy(
            x_hbm.at[f, pl.ds(col, ROWS_PER_W)],
            idx_v.at[pl.ds(f * ROWS_PER_W, ROWS_PER_W)],
            sem3,
        )
    for f in range(F_HALF, NUM_FIELDS):
        pltpu.make_async_copy(
            x_hbm.at[f, pl.ds(col, ROWS_PER_W)],
            idx_v.at[pl.ds(f * ROWS_PER_W, ROWS_PER_W)],
            sem3,
        ).wait()
    pltpu.sync_copy(bias_hbm, bias_v)
    g2 = pltpu.async_copy(
        table_hbm.at[idx_v.at[pl.ds(IDX_HALF, IDX_PER_W - IDX_HALF)]],
        vals_v.at[pl.ds(IDX_HALF, IDX_PER_W - IDX_HALF)],
        sem2,
    )
    bias_vec = bias_v[...]
    g1.wait()

    def accum1(i, _):
        acc0 = bias_vec
        acc1 = vals_v[pl.ds(i * 16, 16)]
        for f in range(1, F_HALF, 2):
            acc0 = acc0 + vals_v[pl.ds(f * ROWS_PER_W + i * 16, 16)]
        for f in range(2, F_HALF, 2):
            acc1 = acc1 + vals_v[pl.ds(f * ROWS_PER_W + i * 16, 16)]
        out_v[pl.ds(i * 16, 16)] = acc0 + acc1
        return 0

    lax.fori_loop(0, ROWS_PER_W // 16, accum1, 0)
    g2.wait()

    def accum2(i, _):
        acc0 = out_v[pl.ds(i * 16, 16)]
        acc1 = vals_v[pl.ds(F_HALF * ROWS_PER_W + i * 16, 16)]
        for f in range(F_HALF + 1, NUM_FIELDS, 2):
            acc0 = acc0 + vals_v[pl.ds(f * ROWS_PER_W + i * 16, 16)]
        for f in range(F_HALF + 2, NUM_FIELDS, 2):
            acc1 = acc1 + vals_v[pl.ds(f * ROWS_PER_W + i * 16, 16)]
        out_v[pl.ds(i * 16, 16)] = acc0 + acc1
        return 0

    lax.fori_loop(0, ROWS_PER_W // 16, accum2, 0)
    pltpu.sync_copy(out_v, out_hbm.at[pl.ds(wid * ROWS_PER_W, ROWS_PER_W)])


def kernel(x, table, bias):
    table_lin = jnp.concatenate(
        [
            table[:TBL_SPLIT].reshape(-1),
            table[TBL_SPLIT:].reshape(-1),
        ]
    )
    xw = x.T
    out = _emb_sum(xw, table_lin, jnp.broadcast_to(bias, (16,)))
    return out.reshape(BATCH, 1)
